# baseline (speedup 1.0000x reference)
import functools
import math

import jax
import jax.numpy as jnp
from jax import lax
from jax.experimental import pallas as pl
from jax.experimental.pallas import tpu as pltpu
from jax.experimental.pallas import tpu_sc as plsc

L = 16
NC = 2
NS = 16
NW = NC * NS
C = 64
ZR = 80


def _proj_body(x_ref, w_ref, asrc_ref, adst_ref, h_ref, as_ref, ad_ref):
    x = x_ref[...]
    h = jnp.dot(x, w_ref[...], preferred_element_type=jnp.float32)
    h_ref[...] = h
    as_ref[...] = jnp.sum(h * asrc_ref[...], axis=1)
    ad_ref[...] = jnp.sum(h * adst_ref[...], axis=1)


def _gat_body(n_nodes, n_edges, cpt,
              h_hbm, eidx_hbm, asrc_hbm, adst_hbm,
              out_hbm, den_hbm,
              acc, asrc_t, adst_t, den_t,
              idx_v0, idx_v1, dsc_v0, dsc_v1,
              w_v0, w_v1, rows_v0, rows_v1,
              sem_i0, sem_i1, sem_r0, sem_r1, sem_s0, sem_s1):
    d = h_hbm.shape[1]
    cid = lax.axis_index("c")
    sid = lax.axis_index("s")
    wid = sid * NC + cid
    nblk = n_nodes // ZR

    def dzero(i, _):
        den_t[0, pl.ds(i * L, L)] = jnp.zeros((L,), jnp.float32)
        return 0
    lax.fori_loop(0, n_nodes // L, dzero, 0)

    def zrow(r, _):
        for j in range(d // L):
            rows_v0[r, pl.ds(j * L, L)] = jnp.zeros((L,), jnp.float32)
        return 0
    lax.fori_loop(0, C, zrow, 0)
    izblk = n_nodes // C
    izrem = n_nodes - izblk * C
    for k in range((izblk + NS - 1) // NS):
        cb = k * NS + sid

        @pl.when(cb < izblk)
        def _():
            pltpu.sync_copy(rows_v0, acc.at[pl.ds(cb * C, C)])
    if izrem:
        @pl.when(sid == 0)
        def _():
            pltpu.sync_copy(rows_v0.at[pl.ds(0, izrem)],
                            acc.at[pl.ds(izblk * C, izrem)])

    pltpu.sync_copy(asrc_hbm, asrc_t)
    pltpu.sync_copy(adst_hbm, adst_t)
    plsc.subcore_barrier()

    idxs = (idx_v0, idx_v1)
    dstscs = (dsc_v0, dsc_v1)
    ws = (w_v0, w_v1)
    rows = (rows_v0, rows_v1)
    sem_idx = (sem_i0, sem_i1)
    sem_row = (sem_r0, sem_r1)
    sem_sc = (sem_s0, sem_s1)
    zero16 = jnp.zeros((L,), jnp.int32)

    def idx_start(t, b):
        base = (wid * cpt + t) * 2 * C
        pltpu.async_copy(eidx_hbm.at[pl.ds(base, 2 * C)], idxs[b],
                         sem_idx[b])

    def idx_wait(t, b):
        base = (wid * cpt + t) * 2 * C
        pltpu.make_async_copy(eidx_hbm.at[pl.ds(base, 2 * C)], idxs[b],
                              sem_idx[b]).wait()

    def compw(t, b, masked):
        base = (wid * cpt + t) * C

        def grp(g, _):
            sidx = idxs[b][pl.ds(g * L, L)]
            didx = idxs[b][pl.ds(C + g * L, L)]
            al = (plsc.load_gather(asrc_t, [sidx])
                  + plsc.load_gather(adst_t, [didx]))
            al = jnp.where(al > 0, al, 0.2 * al)
            if masked:
                eid = base + g * L + lax.iota(jnp.int32, L)
                w = jnp.where(eid < n_edges, jnp.exp(al), 0.0)
            else:
                w = jnp.exp(al)
            ws[b][pl.ds(g * L, L)] = w
            plsc.addupdate_scatter(den_t, [zero16, didx], w)
            return 0
        lax.fori_loop(0, C // L, grp, 0)

    def compw_auto(t, b):
        base = (wid * cpt + t) * C

        @pl.when(base + C <= n_edges)
        def _():
            compw(t, b, False)

        @pl.when(base + C > n_edges)
        def _():
            compw(t, b, True)

    def stage(t, b, nb):
        @pl.when((t >= 1) & (t + 1 < cpt))
        def _():
            pltpu.make_async_copy(rows[nb], acc.at[dstscs[nb]],
                                  sem_sc[nb]).wait()

        @pl.when(t + 1 < cpt)
        def _():
            idx_wait(t + 1, nb)
            pltpu.async_copy(h_hbm.at[idxs[nb].at[pl.ds(0, C)]], rows[nb],
                             sem_row[nb])
            compw_auto(t + 1, nb)

        pltpu.make_async_copy(h_hbm.at[idxs[b].at[pl.ds(0, C)]], rows[b],
                              sem_row[b]).wait()

        for g in range(C // L):
            dstscs[b][pl.ds(g * L, L)] = idxs[b][pl.ds(C + g * L, L)]

        @pl.when(t + 2 < cpt)
        def _():
            idx_start(t + 2, b)

        @plsc.parallel_loop(0, C, 1, unroll=4)
        def _(e):
            wv = plsc.load_gather(ws[b], [jnp.full((L,), e, jnp.int32)])
            for j in range(d // L):
                rows[b][e, pl.ds(j * L, L)] = (
                    rows[b][e, pl.ds(j * L, L)] * wv)

        pltpu.async_copy(rows[b], acc.at[dstscs[b]], sem_sc[b], add=True)

    idx_start(0, 0)
    idx_wait(0, 0)
    pltpu.async_copy(h_hbm.at[idxs[0].at[pl.ds(0, C)]], rows[0], sem_row[0])
    compw_auto(0, 0)
    idx_start(1, 1)

    def pair(i, _):
        stage(i * 2, 0, 1)
        stage(i * 2 + 1, 1, 0)
        return 0
    lax.fori_loop(0, cpt // 2, pair, 0)

    pltpu.make_async_copy(rows[0], acc.at[dstscs[0]], sem_sc[0]).wait()
    pltpu.make_async_copy(rows[1], acc.at[dstscs[1]], sem_sc[1]).wait()
    plsc.subcore_barrier()

    pltpu.sync_copy(den_t, den_hbm.at[wid])
    for k in range((nblk + NS - 1) // NS):
        cb = k * NS + sid

        @pl.when(cb < nblk)
        def _():
            r0 = cb * ZR
            pltpu.sync_copy(acc.at[pl.ds(r0, ZR)],
                            out_hbm.at[cid, pl.ds(r0, ZR)])


def _fin_body(acc_ref, den_ref, x_ref, b_ref, p_ref, o_ref):
    s = acc_ref[0] + acc_ref[1]
    den = jnp.sum(den_ref[...], axis=0)
    o = s / den[:, None] + b_ref[...]
    p = p_ref[0, 0]
    o = jnp.where(o > 0, o, p * o)
    o_ref[...] = o + x_ref[...]


def kernel(x, edge_index, W, att_src, att_dst, bias, prelu_w):
    n, d = x.shape
    e_in = edge_index.shape[1]
    e2 = e_in + n
    cpt = math.ceil(e2 / (NW * C))
    cpt += cpt % 2
    ep = NW * cpt * C

    loop = jnp.arange(n, dtype=edge_index.dtype)
    zpad = jnp.zeros((ep - e2,), edge_index.dtype)
    src = jnp.concatenate([edge_index[0], loop, zpad])
    dst = jnp.concatenate([edge_index[1], loop, zpad])
    nch = ep // C
    eidx = jnp.stack([src.reshape(nch, C), dst.reshape(nch, C)],
                     axis=1).reshape(nch * 2 * C)

    h, a_src_n, a_dst_n = pl.pallas_call(
        _proj_body,
        out_shape=(jax.ShapeDtypeStruct((n, d), jnp.float32),
                   jax.ShapeDtypeStruct((n,), jnp.float32),
                   jax.ShapeDtypeStruct((n,), jnp.float32)),
    )(x, W, att_src, att_dst)

    mesh = plsc.VectorSubcoreMesh(core_axis_name="c", subcore_axis_name="s")
    acc2, den2 = pl.kernel(
        functools.partial(_gat_body, n, e2, cpt),
        out_type=(jax.ShapeDtypeStruct((NC, n, d), jnp.float32),
                  jax.ShapeDtypeStruct((NW, 1, n), jnp.float32)),
        mesh=mesh,
        compiler_params=pltpu.CompilerParams(needs_layout_passes=False),
        scratch_types=[
            pltpu.VMEM_SHARED((n, d), jnp.float32),
            pltpu.VMEM((n,), jnp.float32),
            pltpu.VMEM((n,), jnp.float32),
            pltpu.VMEM((1, n), jnp.float32),
            pltpu.VMEM((2 * C,), jnp.int32),
            pltpu.VMEM((2 * C,), jnp.int32),
            pltpu.VMEM((C,), jnp.int32),
            pltpu.VMEM((C,), jnp.int32),
            pltpu.VMEM((C,), jnp.float32),
            pltpu.VMEM((C,), jnp.float32),
            pltpu.VMEM((C, d), jnp.float32),
            pltpu.VMEM((C, d), jnp.float32),
            pltpu.SemaphoreType.DMA,
            pltpu.SemaphoreType.DMA,
            pltpu.SemaphoreType.DMA,
            pltpu.SemaphoreType.DMA,
            pltpu.SemaphoreType.DMA,
            pltpu.SemaphoreType.DMA,
        ],
    )(h, eidx, a_src_n, a_dst_n)

    blk = 512
    grid = math.ceil(n / blk)
    out = pl.pallas_call(
        _fin_body,
        grid=(grid,),
        in_specs=[
            pl.BlockSpec((2, blk, d), lambda i: (0, i, 0)),
            pl.BlockSpec((NW, blk), lambda i: (0, i)),
            pl.BlockSpec((blk, d), lambda i: (i, 0)),
            pl.BlockSpec((1, d), lambda i: (0, 0)),
            pl.BlockSpec((1, 1), lambda i: (0, 0)),
        ],
        out_specs=pl.BlockSpec((blk, d), lambda i: (i, 0)),
        out_shape=jax.ShapeDtypeStruct((n, d), jnp.float32),
    )(acc2, den2.reshape(NW, n), x, bias.reshape(1, d),
      prelu_w.reshape(1, 1))
    return out


# speedup vs baseline: 1.0968x; 1.0968x over previous
import functools
import math

import jax
import jax.numpy as jnp
from jax import lax
from jax.experimental import pallas as pl
from jax.experimental.pallas import tpu as pltpu
from jax.experimental.pallas import tpu_sc as plsc

L = 16
NC = 2
NS = 16
NW = NC * NS
C = 64
ZR = 80


def _proj_body(x_ref, w_ref, asrc_ref, adst_ref, h_ref, as_ref, ad_ref):
    x = x_ref[...]
    h = jnp.dot(x, w_ref[...], preferred_element_type=jnp.float32)
    h_ref[...] = h
    as_ref[...] = jnp.sum(h * asrc_ref[...], axis=1)
    ad_ref[...] = jnp.sum(h * adst_ref[...], axis=1)


def _gat_body(n_nodes, n_edges, cpt,
              h_hbm, src_hbm, dst_hbm, asrc_hbm, adst_hbm,
              out_hbm, den_hbm,
              acc, asrc_t, adst_t, den_t,
              src_v0, src_v1, dst_v0, dst_v1, dsc_v0, dsc_v1,
              w_v0, w_v1, rows_v0, rows_v1,
              sem_i0, sem_i1, sem_r0, sem_r1, sem_s0, sem_s1):
    d = h_hbm.shape[1]
    cid = lax.axis_index("c")
    sid = lax.axis_index("s")
    wid = sid * NC + cid
    nblk = n_nodes // ZR

    def dzero(i, _):
        den_t[0, pl.ds(i * L, L)] = jnp.zeros((L,), jnp.float32)
        return 0
    lax.fori_loop(0, n_nodes // L, dzero, 0)

    def zrow(r, _):
        for j in range(d // L):
            rows_v0[r, pl.ds(j * L, L)] = jnp.zeros((L,), jnp.float32)
        return 0
    lax.fori_loop(0, C, zrow, 0)
    izblk = n_nodes // C
    izrem = n_nodes - izblk * C
    for k in range((izblk + NS - 1) // NS):
        cb = k * NS + sid

        @pl.when(cb < izblk)
        def _():
            pltpu.sync_copy(rows_v0, acc.at[pl.ds(cb * C, C)])
    if izrem:
        @pl.when(sid == 0)
        def _():
            pltpu.sync_copy(rows_v0.at[pl.ds(0, izrem)],
                            acc.at[pl.ds(izblk * C, izrem)])

    pltpu.sync_copy(asrc_hbm, asrc_t)
    pltpu.sync_copy(adst_hbm, adst_t)
    plsc.subcore_barrier()

    srcs = (src_v0, src_v1)
    dsts = (dst_v0, dst_v1)
    dstscs = (dsc_v0, dsc_v1)
    ws = (w_v0, w_v1)
    rows = (rows_v0, rows_v1)
    sem_idx = (sem_i0, sem_i1)
    sem_row = (sem_r0, sem_r1)
    sem_sc = (sem_s0, sem_s1)
    zero16 = jnp.zeros((L,), jnp.int32)

    def idx_start(t, b):
        base = (wid * cpt + t) * C
        pltpu.async_copy(src_hbm.at[pl.ds(base, C)], srcs[b], sem_idx[b])
        pltpu.async_copy(dst_hbm.at[pl.ds(base, C)], dsts[b], sem_idx[b])

    def idx_wait(t, b):
        base = (wid * cpt + t) * C
        pltpu.make_async_copy(src_hbm.at[pl.ds(base, C)], srcs[b],
                              sem_idx[b]).wait()
        pltpu.make_async_copy(dst_hbm.at[pl.ds(base, C)], dsts[b],
                              sem_idx[b]).wait()

    def compw(t, b):
        base = (wid * cpt + t) * C

        def grp(g, _):
            sidx = srcs[b][pl.ds(g * L, L)]
            didx = dsts[b][pl.ds(g * L, L)]
            al = (plsc.load_gather(asrc_t, [sidx])
                  + plsc.load_gather(adst_t, [didx]))
            al = jnp.where(al > 0, al, 0.2 * al)
            eid = base + g * L + lax.iota(jnp.int32, L)
            w = jnp.where(eid < n_edges, jnp.exp(al), 0.0)
            ws[b][pl.ds(g * L, L)] = w
            plsc.addupdate_scatter(den_t, [zero16, didx], w)
            return 0
        lax.fori_loop(0, C // L, grp, 0)

    def stage(t, b, nb):
        @pl.when((t >= 1) & (t + 1 < cpt))
        def _():
            pltpu.make_async_copy(rows[nb], acc.at[dstscs[nb]],
                                  sem_sc[nb]).wait()

        @pl.when(t + 1 < cpt)
        def _():
            idx_wait(t + 1, nb)
            compw(t + 1, nb)
            pltpu.async_copy(h_hbm.at[srcs[nb]], rows[nb], sem_row[nb])

        pltpu.make_async_copy(h_hbm.at[srcs[b]], rows[b],
                              sem_row[b]).wait()

        for g in range(C // L):
            dstscs[b][pl.ds(g * L, L)] = dsts[b][pl.ds(g * L, L)]

        @pl.when(t + 2 < cpt)
        def _():
            idx_start(t + 2, b)

        @plsc.parallel_loop(0, C, 1, unroll=8)
        def _(e):
            wv = plsc.load_gather(ws[b], [jnp.full((L,), e, jnp.int32)])
            for j in range(d // L):
                rows[b][e, pl.ds(j * L, L)] = (
                    rows[b][e, pl.ds(j * L, L)] * wv)

        pltpu.async_copy(rows[b], acc.at[dstscs[b]], sem_sc[b], add=True)

    idx_start(0, 0)
    idx_wait(0, 0)
    compw(0, 0)
    pltpu.async_copy(h_hbm.at[srcs[0]], rows[0], sem_row[0])
    idx_start(1, 1)

    def pair(i, _):
        stage(i * 2, 0, 1)
        stage(i * 2 + 1, 1, 0)
        return 0
    lax.fori_loop(0, cpt // 2, pair, 0)

    pltpu.make_async_copy(rows[0], acc.at[dstscs[0]], sem_sc[0]).wait()
    pltpu.make_async_copy(rows[1], acc.at[dstscs[1]], sem_sc[1]).wait()
    plsc.subcore_barrier()

    pltpu.sync_copy(den_t, den_hbm.at[wid])
    for k in range((nblk + NS - 1) // NS):
        cb = k * NS + sid

        @pl.when(cb < nblk)
        def _():
            r0 = cb * ZR
            pltpu.sync_copy(acc.at[pl.ds(r0, ZR)],
                            out_hbm.at[cid, pl.ds(r0, ZR)])


def _fin_body(acc_ref, den_ref, x_ref, b_ref, p_ref, o_ref):
    s = acc_ref[0] + acc_ref[1]
    den = jnp.sum(den_ref[...], axis=0)
    o = s / den[:, None] + b_ref[...]
    p = p_ref[0, 0]
    o = jnp.where(o > 0, o, p * o)
    o_ref[...] = o + x_ref[...]


def kernel(x, edge_index, W, att_src, att_dst, bias, prelu_w):
    n, d = x.shape
    e_in = edge_index.shape[1]
    e2 = e_in + n
    cpt = math.ceil(e2 / (NW * C))
    cpt += cpt % 2
    ep = NW * cpt * C

    loop = jnp.arange(n, dtype=edge_index.dtype)
    zpad = jnp.zeros((ep - e2,), edge_index.dtype)
    src = jnp.concatenate([edge_index[0], loop, zpad])
    dst = jnp.concatenate([edge_index[1], loop, zpad])

    h, a_src_n, a_dst_n = pl.pallas_call(
        _proj_body,
        out_shape=(jax.ShapeDtypeStruct((n, d), jnp.float32),
                   jax.ShapeDtypeStruct((n,), jnp.float32),
                   jax.ShapeDtypeStruct((n,), jnp.float32)),
    )(x, W, att_src, att_dst)

    mesh = plsc.VectorSubcoreMesh(core_axis_name="c", subcore_axis_name="s")
    acc2, den2 = pl.kernel(
        functools.partial(_gat_body, n, e2, cpt),
        out_type=(jax.ShapeDtypeStruct((NC, n, d), jnp.float32),
                  jax.ShapeDtypeStruct((NW, 1, n), jnp.float32)),
        mesh=mesh,
        compiler_params=pltpu.CompilerParams(needs_layout_passes=False),
        scratch_types=[
            pltpu.VMEM_SHARED((n, d), jnp.float32),
            pltpu.VMEM((n,), jnp.float32),
            pltpu.VMEM((n,), jnp.float32),
            pltpu.VMEM((1, n), jnp.float32),
            pltpu.VMEM((C,), jnp.int32),
            pltpu.VMEM((C,), jnp.int32),
            pltpu.VMEM((C,), jnp.int32),
            pltpu.VMEM((C,), jnp.int32),
            pltpu.VMEM((C,), jnp.int32),
            pltpu.VMEM((C,), jnp.int32),
            pltpu.VMEM((C,), jnp.float32),
            pltpu.VMEM((C,), jnp.float32),
            pltpu.VMEM((C, d), jnp.float32),
            pltpu.VMEM((C, d), jnp.float32),
            pltpu.SemaphoreType.DMA,
            pltpu.SemaphoreType.DMA,
            pltpu.SemaphoreType.DMA,
            pltpu.SemaphoreType.DMA,
            pltpu.SemaphoreType.DMA,
            pltpu.SemaphoreType.DMA,
        ],
    )(h, src, dst, a_src_n, a_dst_n)

    blk = 512
    grid = math.ceil(n / blk)
    out = pl.pallas_call(
        _fin_body,
        grid=(grid,),
        in_specs=[
            pl.BlockSpec((2, blk, d), lambda i: (0, i, 0)),
            pl.BlockSpec((NW, blk), lambda i: (0, i)),
            pl.BlockSpec((blk, d), lambda i: (i, 0)),
            pl.BlockSpec((1, d), lambda i: (0, 0)),
            pl.BlockSpec((1, 1), lambda i: (0, 0)),
        ],
        out_specs=pl.BlockSpec((blk, d), lambda i: (i, 0)),
        out_shape=jax.ShapeDtypeStruct((n, d), jnp.float32),
    )(acc2, den2.reshape(NW, n), x, bias.reshape(1, d),
      prelu_w.reshape(1, 1))
    return out


# speedup vs baseline: 1.1439x; 1.0429x over previous
import functools
import math

import jax
import jax.numpy as jnp
from jax import lax
from jax.experimental import pallas as pl
from jax.experimental.pallas import tpu as pltpu
from jax.experimental.pallas import tpu_sc as plsc

L = 16
NC = 2
NS = 16
NW = NC * NS
C = 48
NB = 3
ZR = 80


def _proj_body(x_ref, w_ref, asrc_ref, adst_ref, h_ref, as_ref, ad_ref):
    x = x_ref[...]
    h = jnp.dot(x, w_ref[...], preferred_element_type=jnp.float32)
    h_ref[...] = h
    as_ref[...] = jnp.sum(h * asrc_ref[...], axis=1)
    ad_ref[...] = jnp.sum(h * adst_ref[...], axis=1)


def _gat_body(n_nodes, n_edges, cpt,
              h_hbm, src_hbm, dst_hbm, asrc_hbm, adst_hbm,
              out_hbm, den_hbm,
              acc, asrc_t, adst_t, den_t,
              src_v0, src_v1, src_v2, dst_v0, dst_v1, dst_v2,
              dsc_v0, dsc_v1, dsc_v2, w_v0, w_v1, w_v2,
              rows_v0, rows_v1, rows_v2,
              sem_i0, sem_i1, sem_i2, sem_r0, sem_r1, sem_r2,
              sem_s0, sem_s1, sem_s2):
    d = h_hbm.shape[1]
    cid = lax.axis_index("c")
    sid = lax.axis_index("s")
    wid = sid * NC + cid
    nblk = n_nodes // ZR

    def dzero(i, _):
        den_t[0, pl.ds(i * L, L)] = jnp.zeros((L,), jnp.float32)
        return 0
    lax.fori_loop(0, n_nodes // L, dzero, 0)

    def zrow(r, _):
        for j in range(d // L):
            rows_v0[r, pl.ds(j * L, L)] = jnp.zeros((L,), jnp.float32)
        return 0
    lax.fori_loop(0, C, zrow, 0)
    izblk = n_nodes // C
    izrem = n_nodes - izblk * C
    for k in range((izblk + NS - 1) // NS):
        cb = k * NS + sid

        @pl.when(cb < izblk)
        def _():
            pltpu.sync_copy(rows_v0, acc.at[pl.ds(cb * C, C)])
    if izrem:
        @pl.when(sid == 0)
        def _():
            pltpu.sync_copy(rows_v0.at[pl.ds(0, izrem)],
                            acc.at[pl.ds(izblk * C, izrem)])

    pltpu.sync_copy(asrc_hbm, asrc_t)
    pltpu.sync_copy(adst_hbm, adst_t)
    plsc.subcore_barrier()

    srcs = (src_v0, src_v1, src_v2)
    dsts = (dst_v0, dst_v1, dst_v2)
    dstscs = (dsc_v0, dsc_v1, dsc_v2)
    ws = (w_v0, w_v1, w_v2)
    rows = (rows_v0, rows_v1, rows_v2)
    sem_idx = (sem_i0, sem_i1, sem_i2)
    sem_row = (sem_r0, sem_r1, sem_r2)
    sem_sc = (sem_s0, sem_s1, sem_s2)
    zero16 = jnp.zeros((L,), jnp.int32)

    def idx_start(t, b):
        base = (wid * cpt + t) * C
        pltpu.async_copy(src_hbm.at[pl.ds(base, C)], srcs[b], sem_idx[b])
        pltpu.async_copy(dst_hbm.at[pl.ds(base, C)], dsts[b], sem_idx[b])

    def idx_wait(t, b):
        base = (wid * cpt + t) * C
        pltpu.make_async_copy(src_hbm.at[pl.ds(base, C)], srcs[b],
                              sem_idx[b]).wait()
        pltpu.make_async_copy(dst_hbm.at[pl.ds(base, C)], dsts[b],
                              sem_idx[b]).wait()

    def compw(t, b):
        base = (wid * cpt + t) * C

        def grp(g, _):
            sidx = srcs[b][pl.ds(g * L, L)]
            didx = dsts[b][pl.ds(g * L, L)]
            al = (plsc.load_gather(asrc_t, [sidx])
                  + plsc.load_gather(adst_t, [didx]))
            al = jnp.where(al > 0, al, 0.2 * al)
            eid = base + g * L + lax.iota(jnp.int32, L)
            w = jnp.where(eid < n_edges, jnp.exp(al), 0.0)
            ws[b][pl.ds(g * L, L)] = w
            plsc.addupdate_scatter(den_t, [zero16, didx], w)
            return 0
        lax.fori_loop(0, C // L, grp, 0)

    def stage(t, b, n2):
        @pl.when(t + 2 < cpt)
        def _():
            idx_wait(t + 2, n2)
            compw(t + 2, n2)

        pltpu.make_async_copy(h_hbm.at[srcs[b]], rows[b],
                              sem_row[b]).wait()

        for g in range(C // L):
            dstscs[b][pl.ds(g * L, L)] = dsts[b][pl.ds(g * L, L)]

        @pl.when(t + 3 < cpt)
        def _():
            idx_start(t + 3, b)

        @pl.when((t >= 1) & (t + 2 < cpt))
        def _():
            pltpu.make_async_copy(rows[n2], acc.at[dstscs[n2]],
                                  sem_sc[n2]).wait()

        @pl.when(t + 2 < cpt)
        def _():
            pltpu.async_copy(h_hbm.at[srcs[n2]], rows[n2], sem_row[n2])

        @plsc.parallel_loop(0, C, 1, unroll=8)
        def _(e):
            wv = plsc.load_gather(ws[b], [jnp.full((L,), e, jnp.int32)])
            for j in range(d // L):
                rows[b][e, pl.ds(j * L, L)] = (
                    rows[b][e, pl.ds(j * L, L)] * wv)

        pltpu.async_copy(rows[b], acc.at[dstscs[b]], sem_sc[b], add=True)

    idx_start(0, 0)
    idx_wait(0, 0)
    compw(0, 0)
    pltpu.async_copy(h_hbm.at[srcs[0]], rows[0], sem_row[0])
    idx_start(1, 1)
    idx_wait(1, 1)
    compw(1, 1)
    pltpu.async_copy(h_hbm.at[srcs[1]], rows[1], sem_row[1])
    idx_start(2, 2)

    def trio(i, _):
        stage(i * 3, 0, 2)
        stage(i * 3 + 1, 1, 0)
        stage(i * 3 + 2, 2, 1)
        return 0
    lax.fori_loop(0, cpt // NB, trio, 0)

    pltpu.make_async_copy(rows[0], acc.at[dstscs[0]], sem_sc[0]).wait()
    pltpu.make_async_copy(rows[1], acc.at[dstscs[1]], sem_sc[1]).wait()
    pltpu.make_async_copy(rows[2], acc.at[dstscs[2]], sem_sc[2]).wait()
    plsc.subcore_barrier()

    pltpu.sync_copy(den_t, den_hbm.at[wid])
    for k in range((nblk + NS - 1) // NS):
        cb = k * NS + sid

        @pl.when(cb < nblk)
        def _():
            r0 = cb * ZR
            pltpu.sync_copy(acc.at[pl.ds(r0, ZR)],
                            out_hbm.at[cid, pl.ds(r0, ZR)])


def _fin_body(acc_ref, den_ref, x_ref, b_ref, p_ref, o_ref):
    s = acc_ref[0] + acc_ref[1]
    den = jnp.sum(den_ref[...], axis=0)
    o = s / den[:, None] + b_ref[...]
    p = p_ref[0, 0]
    o = jnp.where(o > 0, o, p * o)
    o_ref[...] = o + x_ref[...]


def kernel(x, edge_index, W, att_src, att_dst, bias, prelu_w):
    n, d = x.shape
    e_in = edge_index.shape[1]
    e2 = e_in + n
    cpt = math.ceil(e2 / (NW * C))
    cpt = NB * math.ceil(cpt / NB)
    ep = NW * cpt * C

    loop = jnp.arange(n, dtype=edge_index.dtype)
    zpad = jnp.zeros((ep - e2,), edge_index.dtype)
    src = jnp.concatenate([edge_index[0], loop, zpad])
    dst = jnp.concatenate([edge_index[1], loop, zpad])

    h, a_src_n, a_dst_n = pl.pallas_call(
        _proj_body,
        out_shape=(jax.ShapeDtypeStruct((n, d), jnp.float32),
                   jax.ShapeDtypeStruct((n,), jnp.float32),
                   jax.ShapeDtypeStruct((n,), jnp.float32)),
    )(x, W, att_src, att_dst)

    mesh = plsc.VectorSubcoreMesh(core_axis_name="c", subcore_axis_name="s")
    acc2, den2 = pl.kernel(
        functools.partial(_gat_body, n, e2, cpt),
        out_type=(jax.ShapeDtypeStruct((NC, n, d), jnp.float32),
                  jax.ShapeDtypeStruct((NW, 1, n), jnp.float32)),
        mesh=mesh,
        compiler_params=pltpu.CompilerParams(needs_layout_passes=False),
        scratch_types=[
            pltpu.VMEM_SHARED((n, d), jnp.float32),
            pltpu.VMEM((n,), jnp.float32),
            pltpu.VMEM((n,), jnp.float32),
            pltpu.VMEM((1, n), jnp.float32),
            pltpu.VMEM((C,), jnp.int32),
            pltpu.VMEM((C,), jnp.int32),
            pltpu.VMEM((C,), jnp.int32),
            pltpu.VMEM((C,), jnp.int32),
            pltpu.VMEM((C,), jnp.int32),
            pltpu.VMEM((C,), jnp.int32),
            pltpu.VMEM((C,), jnp.int32),
            pltpu.VMEM((C,), jnp.int32),
            pltpu.VMEM((C,), jnp.int32),
            pltpu.VMEM((C,), jnp.float32),
            pltpu.VMEM((C,), jnp.float32),
            pltpu.VMEM((C,), jnp.float32),
            pltpu.VMEM((C, d), jnp.float32),
            pltpu.VMEM((C, d), jnp.float32),
            pltpu.VMEM((C, d), jnp.float32),
            pltpu.SemaphoreType.DMA,
            pltpu.SemaphoreType.DMA,
            pltpu.SemaphoreType.DMA,
            pltpu.SemaphoreType.DMA,
            pltpu.SemaphoreType.DMA,
            pltpu.SemaphoreType.DMA,
            pltpu.SemaphoreType.DMA,
            pltpu.SemaphoreType.DMA,
            pltpu.SemaphoreType.DMA,
        ],
    )(h, src, dst, a_src_n, a_dst_n)

    blk = 512
    grid = math.ceil(n / blk)
    out = pl.pallas_call(
        _fin_body,
        grid=(grid,),
        in_specs=[
            pl.BlockSpec((2, blk, d), lambda i: (0, i, 0)),
            pl.BlockSpec((NW, blk), lambda i: (0, i)),
            pl.BlockSpec((blk, d), lambda i: (i, 0)),
            pl.BlockSpec((1, d), lambda i: (0, 0)),
            pl.BlockSpec((1, 1), lambda i: (0, 0)),
        ],
        out_specs=pl.BlockSpec((blk, d), lambda i: (i, 0)),
        out_shape=jax.ShapeDtypeStruct((n, d), jnp.float32),
    )(acc2, den2.reshape(NW, n), x, bias.reshape(1, d),
      prelu_w.reshape(1, 1))
    return out
